# Initial kernel scaffold; baseline (speedup 1.0000x reference)
import jax
import jax.numpy as jnp
from jax.experimental import pallas as pl


def kernel(stm_indices, nstm_indices, table, input_bias, W, b):
    raise NotImplementedError("write your pallas kernel here")


# speedup vs baseline: 13.1039x; 13.1039x over previous
import functools

import jax
import jax.numpy as jnp
from jax import lax
from jax.experimental import pallas as pl

N_FEATURES = 768
L1 = 1024
N_BUCKETS = 8
B = 16384
A = 32
TP = 776
BB = 256


def _nnue_block(stm_ref, nstm_ref, tab_ref, bias_ref, w1_ref, w2_ref, b2_ref,
                out_ref):
    iota_f = lax.broadcasted_iota(jnp.int32, (BB, TP), 1)

    def counts_of(idx_ref):
        c = jnp.zeros((BB, TP), jnp.float32)
        for a in range(A):
            col = idx_ref[:, a:a + 1]
            c = c + (col == iota_f).astype(jnp.float32)
        return c

    def half(idx_ref, w_ref):
        c = counts_of(idx_ref)
        emb = jnp.dot(c, tab_ref[...], preferred_element_type=jnp.float32)
        emb = emb + bias_ref[...]
        h = jnp.clip(emb, 0.0, 1.0)
        h = h * h
        return jnp.dot(h, w_ref[...], preferred_element_type=jnp.float32)

    p = half(stm_ref, w1_ref) + half(nstm_ref, w2_ref) + b2_ref[...]

    count = jnp.sum((stm_ref[...] != N_FEATURES).astype(jnp.int32), axis=1,
                    keepdims=True)
    bucket = ((count + 30) // 4) % N_BUCKETS
    sel = (bucket == lax.broadcasted_iota(jnp.int32, (BB, N_BUCKETS), 1))
    out_ref[...] = jnp.sum(p * sel.astype(jnp.float32), axis=1, keepdims=True)


@functools.partial(jax.jit, static_argnames=())
def kernel(stm_indices, nstm_indices, table, input_bias, W, b):
    tab = table.at[N_FEATURES].set(0.0)
    tab = jnp.pad(tab, ((0, TP - (N_FEATURES + 1)), (0, 0)))
    w1 = W[:, :L1].T
    w2 = W[:, L1:].T
    bias2d = input_bias[None, :]
    b2d = b[None, :]
    stm = stm_indices.astype(jnp.int32)
    nstm = nstm_indices.astype(jnp.int32)

    grid = (B // BB,)
    return pl.pallas_call(
        _nnue_block,
        grid=grid,
        in_specs=[
            pl.BlockSpec((BB, A), lambda i: (i, 0)),
            pl.BlockSpec((BB, A), lambda i: (i, 0)),
            pl.BlockSpec((TP, L1), lambda i: (0, 0)),
            pl.BlockSpec((1, L1), lambda i: (0, 0)),
            pl.BlockSpec((L1, N_BUCKETS), lambda i: (0, 0)),
            pl.BlockSpec((L1, N_BUCKETS), lambda i: (0, 0)),
            pl.BlockSpec((1, N_BUCKETS), lambda i: (0, 0)),
        ],
        out_specs=pl.BlockSpec((BB, 1), lambda i: (i, 0)),
        out_shape=jax.ShapeDtypeStruct((B, 1), jnp.float32),
    )(stm, nstm, tab, bias2d, w1, w2, b2d)


# speedup vs baseline: 17.1024x; 1.3051x over previous
import functools

import jax
import jax.numpy as jnp
from jax import lax
from jax.experimental import pallas as pl
from jax.experimental.pallas import tpu as pltpu
from jax.experimental.pallas import tpu_sc as plsc

N_FEATURES = 768
L1 = 1024
N_BUCKETS = 8
B = 16384
A = 32
TP = 776

NC = 2
NS = 16
NW = NC * NS
LANES = 16
RW = B // NW
CC = 64

BB = 256


def _sc_counts_body(stm_hbm, nstm_hbm, zeros_hbm, out_s, out_n, idx_v, cnt_v):
    wid = lax.axis_index("s") * NC + lax.axis_index("c")
    row0 = wid * RW
    ones = jnp.full((LANES,), 1.0, jnp.float32)
    neg_ones = jnp.full((LANES,), -1.0, jnp.float32)
    pltpu.sync_copy(zeros_hbm, cnt_v)

    def side(idx_hbm, out_hbm):
        def chunk(k, carry):
            base = row0 + k * CC

            def scatter(vals):
                def grp(g, c2):
                    off = idx_v[pl.ds(g * LANES, LANES)]
                    plsc.addupdate_scatter(cnt_v, [off], vals)
                    return c2
                lax.fori_loop(0, CC * A // LANES, grp, 0)

            pltpu.sync_copy(idx_hbm.at[pl.ds(base * A, CC * A)], idx_v)
            scatter(ones)
            pltpu.sync_copy(cnt_v, out_hbm.at[pl.ds(base * TP, CC * TP)])
            scatter(neg_ones)
            return carry

        lax.fori_loop(0, RW // CC, chunk, 0)

    side(stm_hbm, out_s)
    side(nstm_hbm, out_n)


@functools.partial(
    pl.kernel,
    out_type=[jax.ShapeDtypeStruct((B * TP,), jnp.float32),
              jax.ShapeDtypeStruct((B * TP,), jnp.float32)],
    mesh=plsc.VectorSubcoreMesh(core_axis_name="c", subcore_axis_name="s"),
    scratch_types=[pltpu.VMEM((CC * A,), jnp.int32),
                   pltpu.VMEM((CC * TP,), jnp.float32)],
    compiler_params=pltpu.CompilerParams(needs_layout_passes=False),
)
def _sc_counts(*args):
    _sc_counts_body(*args)


def _dense_block(cs_ref, cn_ref, stm_ref, tab_ref, bias_ref, w1_ref, w2_ref,
                 b2_ref, out_ref):
    def half(c_ref, w_ref):
        emb = jnp.dot(c_ref[...], tab_ref[...],
                      preferred_element_type=jnp.float32) + bias_ref[...]
        h = jnp.clip(emb, 0.0, 1.0)
        h = h * h
        return jnp.dot(h, w_ref[...], preferred_element_type=jnp.float32)

    p = half(cs_ref, w1_ref) + half(cn_ref, w2_ref) + b2_ref[...]

    count = jnp.sum((stm_ref[...] != N_FEATURES).astype(jnp.int32), axis=1,
                    keepdims=True)
    bucket = ((count + 30) // 4) % N_BUCKETS
    sel = (bucket == lax.broadcasted_iota(jnp.int32, (BB, N_BUCKETS), 1))
    out_ref[...] = jnp.sum(p * sel.astype(jnp.float32), axis=1, keepdims=True)


@jax.jit
def kernel(stm_indices, nstm_indices, table, input_bias, W, b):
    tab = table.at[N_FEATURES].set(0.0)
    tab = jnp.pad(tab, ((0, TP - (N_FEATURES + 1)), (0, 0)))
    w1 = W[:, :L1].T
    w2 = W[:, L1:].T
    bias2d = input_bias[None, :]
    b2d = b[None, :]
    stm = stm_indices.astype(jnp.int32)
    nstm = nstm_indices.astype(jnp.int32)

    zeros = jnp.zeros((CC * TP,), jnp.float32)
    row_off = ((jnp.arange(B * A, dtype=jnp.int32) // A) % CC) * TP
    counts_s, counts_n = _sc_counts(stm.reshape(B * A) + row_off,
                                    nstm.reshape(B * A) + row_off,
                                    zeros)
    counts_s = counts_s.reshape(B, TP)
    counts_n = counts_n.reshape(B, TP)

    grid = (B // BB,)
    return pl.pallas_call(
        _dense_block,
        grid=grid,
        in_specs=[
            pl.BlockSpec((BB, TP), lambda i: (i, 0)),
            pl.BlockSpec((BB, TP), lambda i: (i, 0)),
            pl.BlockSpec((BB, A), lambda i: (i, 0)),
            pl.BlockSpec((TP, L1), lambda i: (0, 0)),
            pl.BlockSpec((1, L1), lambda i: (0, 0)),
            pl.BlockSpec((L1, N_BUCKETS), lambda i: (0, 0)),
            pl.BlockSpec((L1, N_BUCKETS), lambda i: (0, 0)),
            pl.BlockSpec((1, N_BUCKETS), lambda i: (0, 0)),
        ],
        out_specs=pl.BlockSpec((BB, 1), lambda i: (i, 0)),
        out_shape=jax.ShapeDtypeStruct((B, 1), jnp.float32),
    )(counts_s, counts_n, stm, tab, bias2d, w1, w2, b2d)


# speedup vs baseline: 17.6226x; 1.0304x over previous
import functools

import jax
import jax.numpy as jnp
from jax import lax
from jax.experimental import pallas as pl
from jax.experimental.pallas import tpu as pltpu
from jax.experimental.pallas import tpu_sc as plsc

N_FEATURES = 768
L1 = 1024
N_BUCKETS = 8
B = 16384
A = 32
TP = 776

NC = 2
NS = 16
NW = NC * NS
LANES = 16
RW = B // NW
CC = 64

BB = 512


def _sc_counts_body(stm_hbm, nstm_hbm, zeros_hbm, out_s, out_n, idx_v, cnt_v):
    wid = lax.axis_index("s") * NC + lax.axis_index("c")
    row0 = wid * RW
    ones = jnp.full((LANES,), 1.0, jnp.float32)
    neg_ones = jnp.full((LANES,), -1.0, jnp.float32)
    pltpu.sync_copy(zeros_hbm, cnt_v)

    def side(idx_hbm, out_hbm):
        def chunk(k, carry):
            base = row0 + k * CC

            def scatter(vals):
                def grp(g, c2):
                    off = idx_v[pl.ds(g * LANES, LANES)]
                    plsc.addupdate_scatter(cnt_v, [off], vals)
                    return c2
                lax.fori_loop(0, CC * A // LANES, grp, 0)

            pltpu.sync_copy(idx_hbm.at[pl.ds(base * A, CC * A)], idx_v)
            scatter(ones)
            pltpu.sync_copy(cnt_v, out_hbm.at[pl.ds(base * TP, CC * TP)])
            scatter(neg_ones)
            return carry

        lax.fori_loop(0, RW // CC, chunk, 0)

    side(stm_hbm, out_s)
    side(nstm_hbm, out_n)


@functools.partial(
    pl.kernel,
    out_type=[jax.ShapeDtypeStruct((B * TP,), jnp.float32),
              jax.ShapeDtypeStruct((B * TP,), jnp.float32)],
    mesh=plsc.VectorSubcoreMesh(core_axis_name="c", subcore_axis_name="s"),
    scratch_types=[pltpu.VMEM((CC * A,), jnp.int32),
                   pltpu.VMEM((CC * TP,), jnp.float32)],
    compiler_params=pltpu.CompilerParams(needs_layout_passes=False),
)
def _sc_counts(*args):
    _sc_counts_body(*args)


def _dense_block(cs_ref, cn_ref, stm_ref, tab_ref, bias_ref, w1_ref, w2_ref,
                 b2_ref, out_ref):
    def half(c_ref, w_ref):
        c16 = c_ref[...].astype(jnp.bfloat16)
        emb = jnp.dot(c16, tab_ref[...],
                      preferred_element_type=jnp.float32) + bias_ref[...]
        h = jnp.clip(emb, 0.0, 1.0)
        h = h * h
        return jnp.dot(h.astype(jnp.bfloat16), w_ref[...],
                       preferred_element_type=jnp.float32)

    p = half(cs_ref, w1_ref) + half(cn_ref, w2_ref) + b2_ref[...]

    count = jnp.sum((stm_ref[...] != N_FEATURES).astype(jnp.int32), axis=1,
                    keepdims=True)
    bucket = ((count + 30) // 4) % N_BUCKETS
    sel = (bucket == lax.broadcasted_iota(jnp.int32, (BB, N_BUCKETS), 1))
    out_ref[...] = jnp.sum(p * sel.astype(jnp.float32), axis=1, keepdims=True)


@jax.jit
def kernel(stm_indices, nstm_indices, table, input_bias, W, b):
    tab = table.at[N_FEATURES].set(0.0)
    tab = jnp.pad(tab, ((0, TP - (N_FEATURES + 1)), (0, 0)))
    tab = tab.astype(jnp.bfloat16)
    w1 = W[:, :L1].T.astype(jnp.bfloat16)
    w2 = W[:, L1:].T.astype(jnp.bfloat16)
    bias2d = input_bias[None, :]
    b2d = b[None, :]
    stm = stm_indices.astype(jnp.int32)
    nstm = nstm_indices.astype(jnp.int32)

    zeros = jnp.zeros((CC * TP,), jnp.float32)
    row_off = ((jnp.arange(B * A, dtype=jnp.int32) // A) % CC) * TP
    counts_s, counts_n = _sc_counts(stm.reshape(B * A) + row_off,
                                    nstm.reshape(B * A) + row_off,
                                    zeros)
    counts_s = counts_s.reshape(B, TP)
    counts_n = counts_n.reshape(B, TP)

    grid = (B // BB,)
    return pl.pallas_call(
        _dense_block,
        grid=grid,
        in_specs=[
            pl.BlockSpec((BB, TP), lambda i: (i, 0)),
            pl.BlockSpec((BB, TP), lambda i: (i, 0)),
            pl.BlockSpec((BB, A), lambda i: (i, 0)),
            pl.BlockSpec((TP, L1), lambda i: (0, 0)),
            pl.BlockSpec((1, L1), lambda i: (0, 0)),
            pl.BlockSpec((L1, N_BUCKETS), lambda i: (0, 0)),
            pl.BlockSpec((L1, N_BUCKETS), lambda i: (0, 0)),
            pl.BlockSpec((1, N_BUCKETS), lambda i: (0, 0)),
        ],
        out_specs=pl.BlockSpec((BB, 1), lambda i: (i, 0)),
        out_shape=jax.ShapeDtypeStruct((B, 1), jnp.float32),
    )(counts_s, counts_n, stm, tab, bias2d, w1, w2, b2d)
